# Initial kernel scaffold; baseline (speedup 1.0000x reference)
import jax
import jax.numpy as jnp
from jax.experimental import pallas as pl


def kernel(x, edge_index, W_fc, b_fc, W1, b1, W2, b2):
    raise NotImplementedError("write your pallas kernel here")


# speedup vs baseline: 10.4609x; 10.4609x over previous
import functools

import jax
import jax.numpy as jnp
from jax import lax
from jax.experimental import pallas as pl
from jax.experimental.pallas import tpu as pltpu
from jax.experimental.pallas import tpu_sc as plsc

N = 10000
D = 128
E = 320000
NCORES = 2
NSUB = 16
NTILES = NCORES * NSUB
CHUNK = 128
CHUNKS_PER_TILE = 79
EDGES_PER_TILE = CHUNK * CHUNKS_PER_TILE
E_PAD = EDGES_PER_TILE * NTILES
ACC_ROWS = 10240
ROWS_PER_TILE = ACC_ROWS // NSUB
DEG_W = 128

_MESH = plsc.VectorSubcoreMesh(core_axis_name="c", subcore_axis_name="s")


@functools.partial(
    pl.kernel,
    mesh=_MESH,
    out_type=jax.ShapeDtypeStruct((NCORES * ACC_ROWS, DEG_W), jnp.float32),
    scratch_types=[
        pltpu.VMEM((CHUNK,), jnp.int32),
        pltpu.VMEM((CHUNK, DEG_W), jnp.float32),
        pltpu.VMEM((CHUNK, DEG_W), jnp.float32),
        pltpu.VMEM_SHARED((ACC_ROWS, DEG_W), jnp.float32),
    ],
)
def _degree_sc(dst_hbm, ones_hbm, zeros_hbm, out_hbm, dstv, ones_v, wb_v, acc):
    cid = lax.axis_index("c")
    sid = lax.axis_index("s")
    tid = cid * NSUB + sid
    pltpu.sync_copy(zeros_hbm, wb_v)
    for k in range(ROWS_PER_TILE // CHUNK):
        pltpu.sync_copy(wb_v, acc.at[pl.ds(sid * ROWS_PER_TILE + k * CHUNK, CHUNK)])
    pltpu.sync_copy(ones_hbm, ones_v)
    plsc.subcore_barrier()

    def body(i, carry):
        base = tid * EDGES_PER_TILE + i * CHUNK
        pltpu.sync_copy(dst_hbm.at[pl.ds(base, CHUNK)], dstv)
        pltpu.sync_copy(ones_v, acc.at[dstv], add=True)
        return carry

    lax.fori_loop(0, CHUNKS_PER_TILE, body, 0)
    plsc.subcore_barrier()
    for k in range(ROWS_PER_TILE // CHUNK):
        r = sid * ROWS_PER_TILE + k * CHUNK
        pltpu.sync_copy(acc.at[pl.ds(r, CHUNK)], wb_v)
        pltpu.sync_copy(wb_v, out_hbm.at[pl.ds(cid * ACC_ROWS + r, CHUNK)])


@functools.partial(
    pl.kernel,
    mesh=_MESH,
    out_type=jax.ShapeDtypeStruct((NCORES * ACC_ROWS, D), jnp.float32),
    scratch_types=[
        pltpu.VMEM((CHUNK,), jnp.int32),
        pltpu.VMEM((CHUNK,), jnp.int32),
        pltpu.VMEM((CHUNK, D), jnp.float32),
        pltpu.VMEM((CHUNK, D), jnp.float32),
        pltpu.VMEM_SHARED((ACC_ROWS, D), jnp.float32),
        pltpu.SemaphoreType.DMA,
    ],
)
def _scatter_sc(u_hbm, src_hbm, dst_hbm, zeros_hbm, out_hbm,
                srcv, dstv, rows_v, wb_v, acc, sem):
    cid = lax.axis_index("c")
    sid = lax.axis_index("s")
    tid = cid * NSUB + sid
    pltpu.sync_copy(zeros_hbm, wb_v)
    for k in range(ROWS_PER_TILE // CHUNK):
        pltpu.sync_copy(wb_v, acc.at[pl.ds(sid * ROWS_PER_TILE + k * CHUNK, CHUNK)])
    plsc.subcore_barrier()

    def body(i, carry):
        base = tid * EDGES_PER_TILE + i * CHUNK
        pltpu.sync_copy(src_hbm.at[pl.ds(base, CHUNK)], srcv)
        pltpu.sync_copy(dst_hbm.at[pl.ds(base, CHUNK)], dstv)
        pltpu.async_copy(u_hbm.at[srcv], rows_v, sem).wait()
        pltpu.sync_copy(rows_v, acc.at[dstv], add=True)
        return carry

    lax.fori_loop(0, CHUNKS_PER_TILE, body, 0)
    plsc.subcore_barrier()
    for k in range(ROWS_PER_TILE // CHUNK):
        r = sid * ROWS_PER_TILE + k * CHUNK
        pltpu.sync_copy(acc.at[pl.ds(r, CHUNK)], wb_v)
        pltpu.sync_copy(wb_v, out_hbm.at[pl.ds(cid * ACC_ROWS + r, CHUNK)])


BLK = 1000


def _stage_a_body(x_ref, wfc_ref, bfc_ref, w1_ref, deg_ref, u1_ref, dinv_ref):
    d = deg_ref[...]
    deg = d[0] + d[1] + 1.0

    dinvb = jnp.broadcast_to(lax.rsqrt(deg[:, 0:1]), (BLK, D))
    h0 = jnp.maximum(
        jnp.dot(x_ref[...], wfc_ref[...], preferred_element_type=jnp.float32)
        + bfc_ref[...], 0.0)
    u1_ref[...] = jnp.dot(h0, w1_ref[...],
                          preferred_element_type=jnp.float32) * dinvb
    dinv_ref[...] = dinvb


_stage_a = pl.pallas_call(
    _stage_a_body,
    grid=(N // BLK,),
    in_specs=[
        pl.BlockSpec((BLK, D), lambda i: (i, 0)),
        pl.BlockSpec((D, D), lambda i: (0, 0)),
        pl.BlockSpec((1, D), lambda i: (0, 0)),
        pl.BlockSpec((D, D), lambda i: (0, 0)),
        pl.BlockSpec((NCORES, BLK, DEG_W), lambda i: (0, i, 0)),
    ],
    out_specs=[pl.BlockSpec((BLK, D), lambda i: (i, 0))] * 2,
    out_shape=[jax.ShapeDtypeStruct((N, D), jnp.float32)] * 2,
)


def _stage_b_body(s_ref, u1_ref, dinv_ref, b1_ref, w2_ref, u2_ref):
    s = s_ref[...]
    dinvb = dinv_ref[...]
    h1 = jnp.maximum((s[0] + s[1] + u1_ref[...]) * dinvb + b1_ref[...], 0.0)
    u2_ref[...] = jnp.dot(h1, w2_ref[...],
                          preferred_element_type=jnp.float32) * dinvb


_stage_b = pl.pallas_call(
    _stage_b_body,
    grid=(N // BLK,),
    in_specs=[
        pl.BlockSpec((NCORES, BLK, D), lambda i: (0, i, 0)),
        pl.BlockSpec((BLK, D), lambda i: (i, 0)),
        pl.BlockSpec((BLK, D), lambda i: (i, 0)),
        pl.BlockSpec((1, D), lambda i: (0, 0)),
        pl.BlockSpec((D, D), lambda i: (0, 0)),
    ],
    out_specs=pl.BlockSpec((BLK, D), lambda i: (i, 0)),
    out_shape=jax.ShapeDtypeStruct((N, D), jnp.float32),
)


def _stage_c_body(s_ref, u2_ref, dinv_ref, b2_ref, out_ref):
    s = s_ref[...]
    out_ref[...] = (s[0] + s[1] + u2_ref[...]) * dinv_ref[...] + b2_ref[...]


_stage_c = pl.pallas_call(
    _stage_c_body,
    grid=(N // BLK,),
    in_specs=[
        pl.BlockSpec((NCORES, BLK, D), lambda i: (0, i, 0)),
        pl.BlockSpec((BLK, D), lambda i: (i, 0)),
        pl.BlockSpec((BLK, D), lambda i: (i, 0)),
        pl.BlockSpec((1, D), lambda i: (0, 0)),
    ],
    out_specs=pl.BlockSpec((BLK, D), lambda i: (i, 0)),
    out_shape=jax.ShapeDtypeStruct((N, D), jnp.float32),
)


def kernel(x, edge_index, W_fc, b_fc, W1, b1, W2, b2):
    src = edge_index[0].astype(jnp.int32)
    dst = edge_index[1].astype(jnp.int32)
    pad = E_PAD - E
    src_p = jnp.concatenate([src, jnp.zeros((pad,), jnp.int32)])
    dst_p = jnp.concatenate([dst, jnp.full((pad,), N, jnp.int32)])
    ones128 = jnp.ones((CHUNK, DEG_W), jnp.float32)
    zeros128 = jnp.zeros((CHUNK, D), jnp.float32)

    deg = _degree_sc(dst_p, ones128, zeros128).reshape(NCORES, ACC_ROWS, DEG_W)
    u1, dinvb = _stage_a(x, W_fc, b_fc.reshape(1, D), W1, deg)
    s1 = _scatter_sc(u1, src_p, dst_p, zeros128).reshape(NCORES, ACC_ROWS, D)
    u2 = _stage_b(s1, u1, dinvb, b1.reshape(1, D), W2)
    s2 = _scatter_sc(u2, src_p, dst_p, zeros128).reshape(NCORES, ACC_ROWS, D)
    out = _stage_c(s2, u2, dinvb, b2.reshape(1, D))
    return out
